# baseline (speedup 1.0000x reference)
import functools

import jax
import jax.numpy as jnp
import numpy as np
from jax.experimental import pallas as pl
import jax.experimental.pallas.tpu as pltpu

_N = 10000
_NP = 10240
_NS = 2500
_NSP = 2560
_F = 128
_TQ = 256
_C = 256
_NB = _NP // _C
_R2 = 1.0

_HI = jax.lax.Precision.HIGHEST


def _proj_body(x_ref, wr_ref, wo_ref, xr_ref, xo_ref):
    xb = x_ref[...]
    xr_ref[...] = jnp.dot(xb, wr_ref[...], preferred_element_type=jnp.float32,
                          precision=_HI).astype(jnp.bfloat16)
    xo_ref[...] = jnp.dot(xb, wo_ref[...], preferred_element_type=jnp.float32,
                          precision=_HI).astype(jnp.bfloat16)


def _proj(xP, W_rel, W_root):
    blk = 512
    return pl.pallas_call(
        _proj_body,
        grid=(_NP // blk,),
        in_specs=[
            pl.BlockSpec((blk, _F), lambda i: (i, 0)),
            pl.BlockSpec((_F, _F), lambda i: (0, 0)),
            pl.BlockSpec((_F, _F), lambda i: (0, 0)),
        ],
        out_specs=[
            pl.BlockSpec((blk, _F), lambda i: (i, 0)),
            pl.BlockSpec((blk, _F), lambda i: (i, 0)),
        ],
        out_shape=[
            jax.ShapeDtypeStruct((_NP, _F), jnp.bfloat16),
            jax.ShapeDtypeStruct((_NP, _F), jnp.bfloat16),
        ],
    )(xP, W_rel, W_root)


_FR, _FC = 8, _NP // 8


def _fps_body(px_ref, py_ref, pz_ref, psx_ref, psy_ref, psz_ref,
              idx_ref, qx_ref, qy_ref, qz_ref):
    rows = jax.lax.broadcasted_iota(jnp.int32, (_FR, _FC), 0)
    cols = jax.lax.broadcasted_iota(jnp.int32, (_FR, _FC), 1)
    lin = rows * _FC + cols
    flin = lin.astype(jnp.float32)
    real = lin < _N
    dist0 = jnp.where(real, jnp.inf, -jnp.inf).astype(jnp.float32)

    idx_ref[0] = jnp.int32(0)
    sx, sy, sz = psx_ref[0], psy_ref[0], psz_ref[0]
    qx_ref[0] = sx
    qy_ref[0] = sy
    qz_ref[0] = sz

    def body(i, state):
        dist, sx, sy, sz = state
        dx = px_ref[...] - sx
        dy = py_ref[...] - sy
        dz = pz_ref[...] - sz
        d = (dx * dx + dy * dy) + dz * dz
        dist = jnp.minimum(dist, d)
        m = jnp.max(dist, axis=(0, 1), keepdims=True)
        nxt = jnp.min(jnp.where(dist == m, flin, jnp.float32(3e7))
                      ).astype(jnp.int32)
        sx, sy, sz = psx_ref[nxt], psy_ref[nxt], psz_ref[nxt]
        idx_ref[i] = nxt
        qx_ref[i] = sx
        qy_ref[i] = sy
        qz_ref[i] = sz
        return dist, sx, sy, sz

    jax.lax.fori_loop(1, _NS, body, (dist0, sx, sy, sz))


def _fps(px, py, pz, psx, psy, psz):
    sm = functools.partial(pl.BlockSpec, memory_space=pltpu.SMEM)
    return pl.pallas_call(
        _fps_body,
        in_specs=[pl.BlockSpec((_FR, _FC), lambda: (0, 0))] * 3 + [sm()] * 3,
        out_specs=[sm(), sm(), sm(), sm()],
        out_shape=[
            jax.ShapeDtypeStruct((_NS,), jnp.int32),
            jax.ShapeDtypeStruct((_NS,), jnp.float32),
            jax.ShapeDtypeStruct((_NS,), jnp.float32),
            jax.ShapeDtypeStruct((_NS,), jnp.float32),
        ],
    )(px, py, pz, psx, psy, psz)


_CS = 128


def _conv_body(qpos_ref, posT_ref, xr_ref, xo_ref, idxs_ref, b_ref, L_ref,
               out_ref, agg_ref, carry_ref, cmin_ref):
    b = pl.program_id(1)
    base = pl.program_id(0) * _TQ

    @pl.when(b == 0)
    def _init():
        agg_ref[...] = jnp.zeros_like(agg_ref)
        carry_ref[...] = jnp.zeros_like(carry_ref)
        cmin_ref[0, 0] = 0.0

    @pl.when(cmin_ref[0, 0] < 32.0)
    def _aggregate():
        q = qpos_ref[...]
        p = posT_ref[...]
        q2 = jnp.sum(q * q, axis=1, keepdims=True)
        p2 = jnp.sum(p * p, axis=0, keepdims=True)
        qp = jnp.dot(q.astype(jnp.bfloat16), p.astype(jnp.bfloat16),
                     preferred_element_type=jnp.float32)
        d2 = (q2 + p2) - 2.0 * qp
        mf = (d2 <= _R2).astype(jnp.float32)

        carry = carry_ref[...]
        cnt_b = jnp.sum(mf, axis=1, keepdims=True)
        cross = (carry < 32.0) & (carry + cnt_b > 32.0)
        any_cross = jnp.max(cross.astype(jnp.float32))

        @pl.when(any_cross > 0.0)
        def _with_tri():
            parts = []
            run = carry
            for s in range(_C // _CS):
                mfs = mf[:, s * _CS:(s + 1) * _CS]
                excl = jnp.dot(mfs, L_ref[...],
                               preferred_element_type=jnp.float32)
                parts.append(mfs * (run + excl < 32.0).astype(jnp.float32))
                run = run + jnp.sum(mfs, axis=1, keepdims=True)
            A = jnp.concatenate(parts, axis=1).astype(jnp.bfloat16)
            agg_ref[...] += jnp.dot(A, xr_ref[...],
                                    preferred_element_type=jnp.float32)

        @pl.when(any_cross == 0.0)
        def _no_tri():
            A = (mf * (carry < 32.0)).astype(jnp.bfloat16)
            agg_ref[...] += jnp.dot(A, xr_ref[...],
                                    preferred_element_type=jnp.float32)

        carry_ref[...] = carry + cnt_b
        cmin_ref[0, 0] = jnp.min(carry + cnt_b)

    @pl.when(b == _NB - 1)
    def _fin():
        cnt = jnp.minimum(carry_ref[...], 32.0)
        den = jnp.maximum(cnt, 1.0)
        out_ref[...] = agg_ref[...] / den + b_ref[...]
        srows = jax.lax.broadcasted_iota(jnp.int32, (8, _F), 0)

        def grp(g, _):
            rows = []
            for j in range(8):
                r = idxs_ref[base + g * 8 + j]
                blk = xo_ref[pl.ds(8 * (r // 8), 8), :].astype(jnp.float32)
                rows.append(jnp.sum(jnp.where(srows == r % 8, blk, 0.0),
                                    axis=0, keepdims=True))
            out_ref[pl.ds(g * 8, 8), :] += jnp.concatenate(rows, axis=0)
            return 0

        jax.lax.fori_loop(0, _TQ // 8, grp, 0)


def _conv(qposP, posT8, xr, xo, idxS, bias, L):
    return pl.pallas_call(
        _conv_body,
        grid=(_NSP // _TQ, _NB),
        in_specs=[
            pl.BlockSpec((_TQ, 8), lambda t, b: (t, 0)),
            pl.BlockSpec((8, _C), lambda t, b: (0, b)),
            pl.BlockSpec((_C, _F), lambda t, b: (b, 0)),
            pl.BlockSpec((_NP, _F), lambda t, b: (0, 0)),
            pl.BlockSpec(memory_space=pltpu.SMEM),
            pl.BlockSpec((1, _F), lambda t, b: (0, 0)),
            pl.BlockSpec((_CS, _CS), lambda t, b: (0, 0)),
        ],
        out_specs=pl.BlockSpec((_TQ, _F), lambda t, b: (t, 0)),
        out_shape=jax.ShapeDtypeStruct((_NSP, _F), jnp.float32),
        scratch_shapes=[
            pltpu.VMEM((_TQ, _F), jnp.float32),
            pltpu.VMEM((_TQ, 1), jnp.float32),
            pltpu.SMEM((1, 1), jnp.float32),
        ],
    )(qposP, posT8, xr, xo, idxS, bias, L)


def kernel(x, pos, batch, W_rel, b_rel, W_root):
    posP = jnp.pad(pos, ((0, _NP - _N), (0, 0)))
    px = posP[:, 0].reshape(_FR, _FC)
    py = posP[:, 1].reshape(_FR, _FC)
    pz = posP[:, 2].reshape(_FR, _FC)

    xP = jnp.pad(x, ((0, _NP - _N), (0, 0)))
    xr, xo = _proj(xP, W_rel, W_root)

    idx, qx, qy, qz = _fps(px, py, pz, posP[:, 0], posP[:, 1], posP[:, 2])
    qpos = jnp.stack([qx, qy, qz], axis=1)

    posT8 = jnp.zeros((8, _NP), jnp.float32)
    posT8 = posT8.at[:3, :].set(posP.T)
    posT8 = posT8.at[0, _N:].set(1e4)

    qposP = jnp.zeros((_NSP, 8), jnp.float32).at[:_NS, :3].set(qpos)
    idxS = jnp.zeros((_NSP,), jnp.int32).at[:_NS].set(idx)

    L = (jnp.arange(_CS, dtype=jnp.int32)[:, None]
         < jnp.arange(_CS, dtype=jnp.int32)[None, :]).astype(jnp.float32)
    bias = b_rel.reshape(1, _F)

    outP = _conv(qposP, posT8, xr, xo, idxS, bias, L)
    x_out = outP[:_NS]
    qbatch = batch[idx]
    return (x_out, qpos, qbatch, idx)


# speedup vs baseline: 1.0777x; 1.0777x over previous
import functools

import jax
import jax.numpy as jnp
import numpy as np
from jax.experimental import pallas as pl
import jax.experimental.pallas.tpu as pltpu

_N = 10000
_NP = 10240
_NS = 2500
_NSP = 2560
_F = 128
_TQ = 256
_C = 256
_NB = _NP // _C
_R2 = 1.0

_HI = jax.lax.Precision.HIGHEST


def _proj_body(x_ref, wr_ref, wo_ref, xr_ref, xo_ref):
    xb = x_ref[...]
    xr_ref[...] = jnp.dot(xb, wr_ref[...], preferred_element_type=jnp.float32,
                          precision=_HI).astype(jnp.bfloat16)
    xo_ref[...] = jnp.dot(xb, wo_ref[...], preferred_element_type=jnp.float32,
                          precision=_HI).astype(jnp.bfloat16)


def _proj(xP, W_rel, W_root):
    blk = 512
    return pl.pallas_call(
        _proj_body,
        grid=(_NP // blk,),
        in_specs=[
            pl.BlockSpec((blk, _F), lambda i: (i, 0)),
            pl.BlockSpec((_F, _F), lambda i: (0, 0)),
            pl.BlockSpec((_F, _F), lambda i: (0, 0)),
        ],
        out_specs=[
            pl.BlockSpec((blk, _F), lambda i: (i, 0)),
            pl.BlockSpec((blk, _F), lambda i: (i, 0)),
        ],
        out_shape=[
            jax.ShapeDtypeStruct((_NP, _F), jnp.bfloat16),
            jax.ShapeDtypeStruct((_NP, _F), jnp.bfloat16),
        ],
    )(xP, W_rel, W_root)


_FR, _FC = 8, _NP // 8


def _fps_body(px_ref, py_ref, pz_ref, psx_ref, psy_ref, psz_ref,
              idx_ref, qx_ref, qy_ref, qz_ref):
    rows = jax.lax.broadcasted_iota(jnp.int32, (_FR, _FC), 0)
    cols = jax.lax.broadcasted_iota(jnp.int32, (_FR, _FC), 1)
    lin = rows * _FC + cols
    flin = lin.astype(jnp.float32)
    real = lin < _N
    dist0 = jnp.where(real, jnp.inf, -jnp.inf).astype(jnp.float32)

    idx_ref[0] = jnp.int32(0)
    sx, sy, sz = psx_ref[0], psy_ref[0], psz_ref[0]
    qx_ref[0] = sx
    qy_ref[0] = sy
    qz_ref[0] = sz

    def body(i, state):
        dist, sx, sy, sz = state
        dx = px_ref[...] - sx
        dy = py_ref[...] - sy
        dz = pz_ref[...] - sz
        d = (dx * dx + dy * dy) + dz * dz
        dist = jnp.minimum(dist, d)
        m = jnp.max(dist, axis=(0, 1), keepdims=True)
        nxt = jnp.min(jnp.where(dist == m, flin, jnp.float32(3e7))
                      ).astype(jnp.int32)
        sx, sy, sz = psx_ref[nxt], psy_ref[nxt], psz_ref[nxt]
        idx_ref[i] = nxt
        qx_ref[i] = sx
        qy_ref[i] = sy
        qz_ref[i] = sz
        return dist, sx, sy, sz

    jax.lax.fori_loop(1, _NS, body, (dist0, sx, sy, sz))


def _fps(px, py, pz, psx, psy, psz):
    sm = functools.partial(pl.BlockSpec, memory_space=pltpu.SMEM)
    return pl.pallas_call(
        _fps_body,
        in_specs=[pl.BlockSpec((_FR, _FC), lambda: (0, 0))] * 3 + [sm()] * 3,
        out_specs=[sm(), sm(), sm(), sm()],
        out_shape=[
            jax.ShapeDtypeStruct((_NS,), jnp.int32),
            jax.ShapeDtypeStruct((_NS,), jnp.float32),
            jax.ShapeDtypeStruct((_NS,), jnp.float32),
            jax.ShapeDtypeStruct((_NS,), jnp.float32),
        ],
    )(px, py, pz, psx, psy, psz)


_CS = 128


def _conv_body(qpos_ref, posT_ref, xr_ref, xo_ref, idxs_ref, b_ref, L_ref,
               out_ref, agg_ref, carry_ref, cmin_ref):
    b = pl.program_id(1)
    base = pl.program_id(0) * _TQ

    @pl.when(b == 0)
    def _init():
        agg_ref[...] = jnp.zeros_like(agg_ref)
        carry_ref[...] = jnp.zeros_like(carry_ref)
        cmin_ref[0, 0] = 0.0

    @pl.when(cmin_ref[0, 0] < 32.0)
    def _aggregate():
        q = qpos_ref[...]
        p = posT_ref[...]
        q2 = jnp.sum(q * q, axis=1, keepdims=True)
        p2 = jnp.sum(p * p, axis=0, keepdims=True)
        qp = jnp.dot(q.astype(jnp.bfloat16), p.astype(jnp.bfloat16),
                     preferred_element_type=jnp.float32)
        d2 = (q2 + p2) - 2.0 * qp
        mf = (d2 <= _R2).astype(jnp.float32)

        carry = carry_ref[...]
        parts = []
        run = carry
        for s in range(_C // _CS):
            mfs = mf[:, s * _CS:(s + 1) * _CS]
            excl = jnp.dot(mfs, L_ref[...], preferred_element_type=jnp.float32)
            parts.append(mfs * (run + excl < 32.0).astype(jnp.float32))
            run = run + jnp.sum(mfs, axis=1, keepdims=True)
        A = jnp.concatenate(parts, axis=1).astype(jnp.bfloat16)

        agg_ref[...] += jnp.dot(A, xr_ref[...],
                                preferred_element_type=jnp.float32)
        carry_ref[...] = run
        cmin_ref[0, 0] = jnp.min(run)

    @pl.when(b == _NB - 1)
    def _fin():
        cnt = jnp.minimum(carry_ref[...], 32.0)
        den = jnp.maximum(cnt, 1.0)
        out_ref[...] = agg_ref[...] / den + b_ref[...]
        srows = jax.lax.broadcasted_iota(jnp.int32, (8, _F), 0)

        def grp(g, _):
            rows = []
            for j in range(8):
                r = idxs_ref[base + g * 8 + j]
                blk = xo_ref[pl.ds(8 * (r // 8), 8), :].astype(jnp.float32)
                rows.append(jnp.sum(jnp.where(srows == r % 8, blk, 0.0),
                                    axis=0, keepdims=True))
            out_ref[pl.ds(g * 8, 8), :] += jnp.concatenate(rows, axis=0)
            return 0

        jax.lax.fori_loop(0, _TQ // 8, grp, 0)


def _conv(qposP, posT8, xr, xo, idxS, bias, L):
    return pl.pallas_call(
        _conv_body,
        grid=(_NSP // _TQ, _NB),
        in_specs=[
            pl.BlockSpec((_TQ, 8), lambda t, b: (t, 0)),
            pl.BlockSpec((8, _C), lambda t, b: (0, b)),
            pl.BlockSpec((_C, _F), lambda t, b: (b, 0)),
            pl.BlockSpec((_NP, _F), lambda t, b: (0, 0)),
            pl.BlockSpec(memory_space=pltpu.SMEM),
            pl.BlockSpec((1, _F), lambda t, b: (0, 0)),
            pl.BlockSpec((_CS, _CS), lambda t, b: (0, 0)),
        ],
        out_specs=pl.BlockSpec((_TQ, _F), lambda t, b: (t, 0)),
        out_shape=jax.ShapeDtypeStruct((_NSP, _F), jnp.float32),
        scratch_shapes=[
            pltpu.VMEM((_TQ, _F), jnp.float32),
            pltpu.VMEM((_TQ, 1), jnp.float32),
            pltpu.SMEM((1, 1), jnp.float32),
        ],
    )(qposP, posT8, xr, xo, idxS, bias, L)


def kernel(x, pos, batch, W_rel, b_rel, W_root):
    posP = jnp.pad(pos, ((0, _NP - _N), (0, 0)))
    px = posP[:, 0].reshape(_FR, _FC)
    py = posP[:, 1].reshape(_FR, _FC)
    pz = posP[:, 2].reshape(_FR, _FC)

    xP = jnp.pad(x, ((0, _NP - _N), (0, 0)))
    xr, xo = _proj(xP, W_rel, W_root)

    idx, qx, qy, qz = _fps(px, py, pz, posP[:, 0], posP[:, 1], posP[:, 2])
    qpos = jnp.stack([qx, qy, qz], axis=1)

    posT8 = jnp.zeros((8, _NP), jnp.float32)
    posT8 = posT8.at[:3, :].set(posP.T)
    posT8 = posT8.at[0, _N:].set(1e4)

    qposP = jnp.zeros((_NSP, 8), jnp.float32).at[:_NS, :3].set(qpos)
    idxS = jnp.zeros((_NSP,), jnp.int32).at[:_NS].set(idx)

    L = (jnp.arange(_CS, dtype=jnp.int32)[:, None]
         < jnp.arange(_CS, dtype=jnp.int32)[None, :]).astype(jnp.float32)
    bias = b_rel.reshape(1, _F)

    outP = _conv(qposP, posT8, xr, xo, idxS, bias, L)
    x_out = outP[:_NS]
    qbatch = batch[idx]
    return (x_out, qpos, qbatch, idx)


# speedup vs baseline: 1.0913x; 1.0125x over previous
import functools

import jax
import jax.numpy as jnp
import numpy as np
from jax.experimental import pallas as pl
import jax.experimental.pallas.tpu as pltpu
from jax.experimental.pallas import tpu_sc as plsc

_N = 10000
_NP = 10240
_NS = 2500
_NSP = 2560
_F = 128
_TQ = 256
_C = 256
_NB = _NP // _C
_R2 = 1.0

_HI = jax.lax.Precision.HIGHEST


def _proj_body(x_ref, wr_ref, wo_ref, xr_ref, xo_ref):
    xb = x_ref[...]
    xr_ref[...] = jnp.dot(xb, wr_ref[...], preferred_element_type=jnp.float32,
                          precision=_HI).astype(jnp.bfloat16)
    xo_ref[...] = jnp.dot(xb, wo_ref[...], preferred_element_type=jnp.float32,
                          precision=_HI)


def _proj(xP, W_rel, W_root):
    blk = 512
    return pl.pallas_call(
        _proj_body,
        grid=(_NP // blk,),
        in_specs=[
            pl.BlockSpec((blk, _F), lambda i: (i, 0)),
            pl.BlockSpec((_F, _F), lambda i: (0, 0)),
            pl.BlockSpec((_F, _F), lambda i: (0, 0)),
        ],
        out_specs=[
            pl.BlockSpec((blk, _F), lambda i: (i, 0)),
            pl.BlockSpec((blk, _F), lambda i: (i, 0)),
        ],
        out_shape=[
            jax.ShapeDtypeStruct((_NP, _F), jnp.bfloat16),
            jax.ShapeDtypeStruct((_NP, _F), jnp.float32),
        ],
    )(xP, W_rel, W_root)


def _sc_gather(xo, idxS):
    info = plsc.get_sparse_core_info()
    nw = info.num_cores * info.num_subcores
    bpw = _NSP // nw
    mesh = plsc.VectorSubcoreMesh(core_axis_name="c", subcore_axis_name="s")

    @functools.partial(
        pl.kernel, mesh=mesh,
        out_type=jax.ShapeDtypeStruct((_NSP, _F), jnp.float32),
        scratch_types=[
            pltpu.VMEM((bpw,), jnp.int32),
            pltpu.VMEM((bpw, _F), jnp.float32),
            pltpu.SemaphoreType.DMA,
        ],
    )
    def k(table_hbm, idx_hbm, out_hbm, idx_v, rows_v, sem):
        wid = (jax.lax.axis_index("s") * info.num_cores
               + jax.lax.axis_index("c"))
        base = wid * bpw
        pltpu.sync_copy(idx_hbm.at[pl.ds(base, bpw)], idx_v)
        pltpu.async_copy(table_hbm.at[idx_v], rows_v, sem).wait()
        pltpu.sync_copy(rows_v, out_hbm.at[pl.ds(base, bpw)])

    return k(xo, idxS)


_FR, _FC = 8, _NP // 8


def _fps_body(px_ref, py_ref, pz_ref, psx_ref, psy_ref, psz_ref,
              idx_ref, qx_ref, qy_ref, qz_ref):
    rows = jax.lax.broadcasted_iota(jnp.int32, (_FR, _FC), 0)
    cols = jax.lax.broadcasted_iota(jnp.int32, (_FR, _FC), 1)
    lin = rows * _FC + cols
    flin = lin.astype(jnp.float32)
    real = lin < _N
    dist0 = jnp.where(real, jnp.inf, -jnp.inf).astype(jnp.float32)

    idx_ref[0] = jnp.int32(0)
    sx, sy, sz = psx_ref[0], psy_ref[0], psz_ref[0]
    qx_ref[0] = sx
    qy_ref[0] = sy
    qz_ref[0] = sz

    def body(i, state):
        dist, sx, sy, sz = state
        dx = px_ref[...] - sx
        dy = py_ref[...] - sy
        dz = pz_ref[...] - sz
        d = (dx * dx + dy * dy) + dz * dz
        dist = jnp.minimum(dist, d)
        m = jnp.max(dist, axis=(0, 1), keepdims=True)
        nxt = jnp.min(jnp.where(dist == m, flin, jnp.float32(3e7))
                      ).astype(jnp.int32)
        sx, sy, sz = psx_ref[nxt], psy_ref[nxt], psz_ref[nxt]
        idx_ref[i] = nxt
        qx_ref[i] = sx
        qy_ref[i] = sy
        qz_ref[i] = sz
        return dist, sx, sy, sz

    jax.lax.fori_loop(1, _NS, body, (dist0, sx, sy, sz))


def _fps(px, py, pz, psx, psy, psz):
    sm = functools.partial(pl.BlockSpec, memory_space=pltpu.SMEM)
    return pl.pallas_call(
        _fps_body,
        in_specs=[pl.BlockSpec((_FR, _FC), lambda: (0, 0))] * 3 + [sm()] * 3,
        out_specs=[sm(), sm(), sm(), sm()],
        out_shape=[
            jax.ShapeDtypeStruct((_NS,), jnp.int32),
            jax.ShapeDtypeStruct((_NS,), jnp.float32),
            jax.ShapeDtypeStruct((_NS,), jnp.float32),
            jax.ShapeDtypeStruct((_NS,), jnp.float32),
        ],
    )(px, py, pz, psx, psy, psz)


_CS = 128


def _conv_body(qpos_ref, posT_ref, xr_ref, b_ref, L_ref,
               out_ref, agg_ref, carry_ref, cmin_ref):
    b = pl.program_id(1)

    @pl.when(b == 0)
    def _init():
        agg_ref[...] = jnp.zeros_like(agg_ref)
        carry_ref[...] = jnp.zeros_like(carry_ref)
        cmin_ref[0, 0] = 0.0

    @pl.when(cmin_ref[0, 0] < 32.0)
    def _aggregate():
        q = qpos_ref[...]
        p = posT_ref[...]
        q2 = jnp.sum(q * q, axis=1, keepdims=True)
        p2 = jnp.sum(p * p, axis=0, keepdims=True)
        qp = jnp.dot(q.astype(jnp.bfloat16), p.astype(jnp.bfloat16),
                     preferred_element_type=jnp.float32)
        d2 = (q2 + p2) - 2.0 * qp
        mf = (d2 <= _R2).astype(jnp.float32)

        carry = carry_ref[...]
        parts = []
        run = carry
        for s in range(_C // _CS):
            mfs = mf[:, s * _CS:(s + 1) * _CS]
            excl = jnp.dot(mfs, L_ref[...], preferred_element_type=jnp.float32)
            parts.append(mfs * (run + excl < 32.0).astype(jnp.float32))
            run = run + jnp.sum(mfs, axis=1, keepdims=True)
        A = jnp.concatenate(parts, axis=1).astype(jnp.bfloat16)

        agg_ref[...] += jnp.dot(A, xr_ref[...],
                                preferred_element_type=jnp.float32)
        carry_ref[...] = run
        cmin_ref[0, 0] = jnp.min(run)

    @pl.when(b == _NB - 1)
    def _fin():
        cnt = jnp.minimum(carry_ref[...], 32.0)
        den = jnp.maximum(cnt, 1.0)
        out_ref[...] = agg_ref[...] / den + b_ref[...]


def _conv(qposP, posT8, xr, bias, L):
    return pl.pallas_call(
        _conv_body,
        grid=(_NSP // _TQ, _NB),
        in_specs=[
            pl.BlockSpec((_TQ, 8), lambda t, b: (t, 0)),
            pl.BlockSpec((8, _C), lambda t, b: (0, b)),
            pl.BlockSpec((_C, _F), lambda t, b: (b, 0)),
            pl.BlockSpec((1, _F), lambda t, b: (0, 0)),
            pl.BlockSpec((_CS, _CS), lambda t, b: (0, 0)),
        ],
        out_specs=pl.BlockSpec((_TQ, _F), lambda t, b: (t, 0)),
        out_shape=jax.ShapeDtypeStruct((_NSP, _F), jnp.float32),
        scratch_shapes=[
            pltpu.VMEM((_TQ, _F), jnp.float32),
            pltpu.VMEM((_TQ, 1), jnp.float32),
            pltpu.SMEM((1, 1), jnp.float32),
        ],
    )(qposP, posT8, xr, bias, L)


def kernel(x, pos, batch, W_rel, b_rel, W_root):
    posP = jnp.pad(pos, ((0, _NP - _N), (0, 0)))
    px = posP[:, 0].reshape(_FR, _FC)
    py = posP[:, 1].reshape(_FR, _FC)
    pz = posP[:, 2].reshape(_FR, _FC)

    xP = jnp.pad(x, ((0, _NP - _N), (0, 0)))
    xr, xo = _proj(xP, W_rel, W_root)

    idx, qx, qy, qz = _fps(px, py, pz, posP[:, 0], posP[:, 1], posP[:, 2])
    qpos = jnp.stack([qx, qy, qz], axis=1)

    posT8 = jnp.zeros((8, _NP), jnp.float32)
    posT8 = posT8.at[:3, :].set(posP.T)
    posT8 = posT8.at[0, _N:].set(1e4)

    qposP = jnp.zeros((_NSP, 8), jnp.float32).at[:_NS, :3].set(qpos)
    idxS = jnp.zeros((_NSP,), jnp.int32).at[:_NS].set(idx)

    L = (jnp.arange(_CS, dtype=jnp.int32)[:, None]
         < jnp.arange(_CS, dtype=jnp.int32)[None, :]).astype(jnp.float32)
    bias = b_rel.reshape(1, _F)

    xsel = _sc_gather(xo, idxS)
    outP = _conv(qposP, posT8, xr, bias, L)
    x_out = (outP + xsel)[:_NS]
    qbatch = batch[idx]
    return (x_out, qpos, qbatch, idx)


# speedup vs baseline: 1.2552x; 1.1503x over previous
import functools

import jax
import jax.numpy as jnp
import numpy as np
from jax.experimental import pallas as pl
import jax.experimental.pallas.tpu as pltpu
from jax.experimental.pallas import tpu_sc as plsc

_N = 10000
_NP = 10240
_NS = 2500
_NSP = 2560
_F = 128
_TQ = 256
_C = 512
_NB = _NP // _C
_R2 = 1.0

_HI = jax.lax.Precision.HIGHEST


def _proj_body(x_ref, wr_ref, wo_ref, xr_ref, xo_ref):
    xb = x_ref[...]
    xr_ref[...] = jnp.dot(xb, wr_ref[...], preferred_element_type=jnp.float32,
                          precision=_HI).astype(jnp.bfloat16)
    xo_ref[...] = jnp.dot(xb, wo_ref[...], preferred_element_type=jnp.float32,
                          precision=_HI)


def _proj(xP, W_rel, W_root):
    blk = 512
    return pl.pallas_call(
        _proj_body,
        grid=(_NP // blk,),
        in_specs=[
            pl.BlockSpec((blk, _F), lambda i: (i, 0)),
            pl.BlockSpec((_F, _F), lambda i: (0, 0)),
            pl.BlockSpec((_F, _F), lambda i: (0, 0)),
        ],
        out_specs=[
            pl.BlockSpec((blk, _F), lambda i: (i, 0)),
            pl.BlockSpec((blk, _F), lambda i: (i, 0)),
        ],
        out_shape=[
            jax.ShapeDtypeStruct((_NP, _F), jnp.bfloat16),
            jax.ShapeDtypeStruct((_NP, _F), jnp.float32),
        ],
    )(xP, W_rel, W_root)


def _sc_gather(xo, idxS):
    info = plsc.get_sparse_core_info()
    nw = info.num_cores * info.num_subcores
    bpw = _NSP // nw
    mesh = plsc.VectorSubcoreMesh(core_axis_name="c", subcore_axis_name="s")

    @functools.partial(
        pl.kernel, mesh=mesh,
        out_type=jax.ShapeDtypeStruct((_NSP, _F), jnp.float32),
        scratch_types=[
            pltpu.VMEM((bpw,), jnp.int32),
            pltpu.VMEM((bpw, _F), jnp.float32),
            pltpu.SemaphoreType.DMA,
        ],
    )
    def k(table_hbm, idx_hbm, out_hbm, idx_v, rows_v, sem):
        wid = (jax.lax.axis_index("s") * info.num_cores
               + jax.lax.axis_index("c"))
        base = wid * bpw
        pltpu.sync_copy(idx_hbm.at[pl.ds(base, bpw)], idx_v)
        pltpu.async_copy(table_hbm.at[idx_v], rows_v, sem).wait()
        pltpu.sync_copy(rows_v, out_hbm.at[pl.ds(base, bpw)])

    return k(xo, idxS)


_FR, _FC = 8, _NP // 8


def _fps_body(px_ref, py_ref, pz_ref, psx_ref, psy_ref, psz_ref,
              idx_ref, qx_ref, qy_ref, qz_ref):
    rows = jax.lax.broadcasted_iota(jnp.int32, (_FR, _FC), 0)
    cols = jax.lax.broadcasted_iota(jnp.int32, (_FR, _FC), 1)
    lin = rows * _FC + cols
    flin = lin.astype(jnp.float32)
    real = lin < _N
    dist0 = jnp.where(real, jnp.inf, -jnp.inf).astype(jnp.float32)

    idx_ref[0] = jnp.int32(0)
    sx, sy, sz = psx_ref[0], psy_ref[0], psz_ref[0]
    qx_ref[0] = sx
    qy_ref[0] = sy
    qz_ref[0] = sz

    def body(i, state):
        dist, sx, sy, sz = state
        dx = px_ref[...] - sx
        dy = py_ref[...] - sy
        dz = pz_ref[...] - sz
        d = (dx * dx + dy * dy) + dz * dz
        dist = jnp.minimum(dist, d)
        m = jnp.max(dist, axis=(0, 1), keepdims=True)
        nxt = jnp.min(jnp.where(dist == m, flin, jnp.float32(3e7))
                      ).astype(jnp.int32)
        sx, sy, sz = psx_ref[nxt], psy_ref[nxt], psz_ref[nxt]
        idx_ref[i] = nxt
        qx_ref[i] = sx
        qy_ref[i] = sy
        qz_ref[i] = sz
        return dist, sx, sy, sz

    jax.lax.fori_loop(1, _NS, body, (dist0, sx, sy, sz))


def _fps(px, py, pz, psx, psy, psz):
    sm = functools.partial(pl.BlockSpec, memory_space=pltpu.SMEM)
    return pl.pallas_call(
        _fps_body,
        in_specs=[pl.BlockSpec((_FR, _FC), lambda: (0, 0))] * 3 + [sm()] * 3,
        out_specs=[sm(), sm(), sm(), sm()],
        out_shape=[
            jax.ShapeDtypeStruct((_NS,), jnp.int32),
            jax.ShapeDtypeStruct((_NS,), jnp.float32),
            jax.ShapeDtypeStruct((_NS,), jnp.float32),
            jax.ShapeDtypeStruct((_NS,), jnp.float32),
        ],
    )(px, py, pz, psx, psy, psz)


_CS = 128


def _conv_body(qpos_ref, posT_ref, xr_ref, b_ref, L_ref,
               out_ref, agg_ref, carry_ref, cmin_ref):
    b = pl.program_id(1)

    @pl.when(b == 0)
    def _init():
        agg_ref[...] = jnp.zeros_like(agg_ref)
        carry_ref[...] = jnp.zeros_like(carry_ref)
        cmin_ref[0, 0] = 0.0

    @pl.when(cmin_ref[0, 0] < 32.0)
    def _aggregate():
        q = qpos_ref[...]
        p = posT_ref[...]
        q2 = jnp.sum(q * q, axis=1, keepdims=True)
        p2 = jnp.sum(p * p, axis=0, keepdims=True)
        qp = jnp.dot(q.astype(jnp.bfloat16), p.astype(jnp.bfloat16),
                     preferred_element_type=jnp.float32)
        d2 = (q2 + p2) - 2.0 * qp
        mf = (d2 <= _R2).astype(jnp.float32)

        carry = carry_ref[...]
        parts = []
        run = carry
        for s in range(_C // _CS):
            mfs = mf[:, s * _CS:(s + 1) * _CS]
            excl = jnp.dot(mfs, L_ref[...], preferred_element_type=jnp.float32)
            parts.append(mfs * (run + excl < 32.0).astype(jnp.float32))
            run = run + jnp.sum(mfs, axis=1, keepdims=True)
        A = jnp.concatenate(parts, axis=1).astype(jnp.bfloat16)

        agg_ref[...] += jnp.dot(A, xr_ref[...],
                                preferred_element_type=jnp.float32)
        carry_ref[...] = run
        cmin_ref[0, 0] = jnp.min(run)

    @pl.when(b == _NB - 1)
    def _fin():
        cnt = jnp.minimum(carry_ref[...], 32.0)
        den = jnp.maximum(cnt, 1.0)
        out_ref[...] = agg_ref[...] / den + b_ref[...]


def _conv(qposP, posT8, xr, bias, L):
    return pl.pallas_call(
        _conv_body,
        grid=(_NSP // _TQ, _NB),
        in_specs=[
            pl.BlockSpec((_TQ, 8), lambda t, b: (t, 0)),
            pl.BlockSpec((8, _C), lambda t, b: (0, b)),
            pl.BlockSpec((_C, _F), lambda t, b: (b, 0)),
            pl.BlockSpec((1, _F), lambda t, b: (0, 0)),
            pl.BlockSpec((_CS, _CS), lambda t, b: (0, 0)),
        ],
        out_specs=pl.BlockSpec((_TQ, _F), lambda t, b: (t, 0)),
        out_shape=jax.ShapeDtypeStruct((_NSP, _F), jnp.float32),
        scratch_shapes=[
            pltpu.VMEM((_TQ, _F), jnp.float32),
            pltpu.VMEM((_TQ, 1), jnp.float32),
            pltpu.SMEM((1, 1), jnp.float32),
        ],
    )(qposP, posT8, xr, bias, L)


def kernel(x, pos, batch, W_rel, b_rel, W_root):
    posP = jnp.pad(pos, ((0, _NP - _N), (0, 0)))
    px = posP[:, 0].reshape(_FR, _FC)
    py = posP[:, 1].reshape(_FR, _FC)
    pz = posP[:, 2].reshape(_FR, _FC)

    xP = jnp.pad(x, ((0, _NP - _N), (0, 0)))
    xr, xo = _proj(xP, W_rel, W_root)

    idx, qx, qy, qz = _fps(px, py, pz, posP[:, 0], posP[:, 1], posP[:, 2])
    qpos = jnp.stack([qx, qy, qz], axis=1)

    posT8 = jnp.zeros((8, _NP), jnp.float32)
    posT8 = posT8.at[:3, :].set(posP.T)
    posT8 = posT8.at[0, _N:].set(1e4)

    qposP = jnp.zeros((_NSP, 8), jnp.float32).at[:_NS, :3].set(qpos)
    idxS = jnp.zeros((_NSP,), jnp.int32).at[:_NS].set(idx)

    L = (jnp.arange(_CS, dtype=jnp.int32)[:, None]
         < jnp.arange(_CS, dtype=jnp.int32)[None, :]).astype(jnp.float32)
    bias = b_rel.reshape(1, _F)

    xsel = _sc_gather(xo, idxS)
    outP = _conv(qposP, posT8, xr, bias, L)
    x_out = (outP + xsel)[:_NS]
    qbatch = batch[idx]
    return (x_out, qpos, qbatch, idx)


# speedup vs baseline: 1.3265x; 1.0568x over previous
import functools

import jax
import jax.numpy as jnp
import numpy as np
from jax.experimental import pallas as pl
import jax.experimental.pallas.tpu as pltpu
from jax.experimental.pallas import tpu_sc as plsc

_N = 10000
_NP = 10240
_NS = 2500
_NSP = 2560
_F = 128
_TQ = 256
_C = 1024
_NB = _NP // _C
_R2 = 1.0

_HI = jax.lax.Precision.HIGHEST


def _proj_body(x_ref, wr_ref, wo_ref, xr_ref, xo_ref):
    xb = x_ref[...]
    xr_ref[...] = jnp.dot(xb, wr_ref[...], preferred_element_type=jnp.float32,
                          precision=_HI).astype(jnp.bfloat16)
    xo_ref[...] = jnp.dot(xb, wo_ref[...], preferred_element_type=jnp.float32,
                          precision=_HI)


def _proj(xP, W_rel, W_root):
    blk = 512
    return pl.pallas_call(
        _proj_body,
        grid=(_NP // blk,),
        in_specs=[
            pl.BlockSpec((blk, _F), lambda i: (i, 0)),
            pl.BlockSpec((_F, _F), lambda i: (0, 0)),
            pl.BlockSpec((_F, _F), lambda i: (0, 0)),
        ],
        out_specs=[
            pl.BlockSpec((blk, _F), lambda i: (i, 0)),
            pl.BlockSpec((blk, _F), lambda i: (i, 0)),
        ],
        out_shape=[
            jax.ShapeDtypeStruct((_NP, _F), jnp.bfloat16),
            jax.ShapeDtypeStruct((_NP, _F), jnp.float32),
        ],
    )(xP, W_rel, W_root)


def _sc_gather(xo, idxS):
    info = plsc.get_sparse_core_info()
    nw = info.num_cores * info.num_subcores
    bpw = _NSP // nw
    mesh = plsc.VectorSubcoreMesh(core_axis_name="c", subcore_axis_name="s")

    @functools.partial(
        pl.kernel, mesh=mesh,
        out_type=jax.ShapeDtypeStruct((_NSP, _F), jnp.float32),
        scratch_types=[
            pltpu.VMEM((bpw,), jnp.int32),
            pltpu.VMEM((bpw, _F), jnp.float32),
            pltpu.SemaphoreType.DMA,
        ],
    )
    def k(table_hbm, idx_hbm, out_hbm, idx_v, rows_v, sem):
        wid = (jax.lax.axis_index("s") * info.num_cores
               + jax.lax.axis_index("c"))
        base = wid * bpw
        pltpu.sync_copy(idx_hbm.at[pl.ds(base, bpw)], idx_v)
        pltpu.async_copy(table_hbm.at[idx_v], rows_v, sem).wait()
        pltpu.sync_copy(rows_v, out_hbm.at[pl.ds(base, bpw)])

    return k(xo, idxS)


_FR, _FC = 8, _NP // 8


def _fps_body(px_ref, py_ref, pz_ref, psx_ref, psy_ref, psz_ref,
              idx_ref, qx_ref, qy_ref, qz_ref):
    rows = jax.lax.broadcasted_iota(jnp.int32, (_FR, _FC), 0)
    cols = jax.lax.broadcasted_iota(jnp.int32, (_FR, _FC), 1)
    lin = rows * _FC + cols
    flin = lin.astype(jnp.float32)
    real = lin < _N
    dist0 = jnp.where(real, jnp.inf, -jnp.inf).astype(jnp.float32)

    idx_ref[0] = jnp.int32(0)
    sx, sy, sz = psx_ref[0], psy_ref[0], psz_ref[0]
    qx_ref[0] = sx
    qy_ref[0] = sy
    qz_ref[0] = sz

    def body(i, state):
        dist, sx, sy, sz = state
        dx = px_ref[...] - sx
        dy = py_ref[...] - sy
        dz = pz_ref[...] - sz
        d = (dx * dx + dy * dy) + dz * dz
        dist = jnp.minimum(dist, d)
        m = jnp.max(dist, axis=(0, 1), keepdims=True)
        nxt = jnp.min(jnp.where(dist == m, flin, jnp.float32(3e7))
                      ).astype(jnp.int32)
        sx, sy, sz = psx_ref[nxt], psy_ref[nxt], psz_ref[nxt]
        idx_ref[i] = nxt
        qx_ref[i] = sx
        qy_ref[i] = sy
        qz_ref[i] = sz
        return dist, sx, sy, sz

    jax.lax.fori_loop(1, _NS, body, (dist0, sx, sy, sz))


def _fps(px, py, pz, psx, psy, psz):
    sm = functools.partial(pl.BlockSpec, memory_space=pltpu.SMEM)
    return pl.pallas_call(
        _fps_body,
        in_specs=[pl.BlockSpec((_FR, _FC), lambda: (0, 0))] * 3 + [sm()] * 3,
        out_specs=[sm(), sm(), sm(), sm()],
        out_shape=[
            jax.ShapeDtypeStruct((_NS,), jnp.int32),
            jax.ShapeDtypeStruct((_NS,), jnp.float32),
            jax.ShapeDtypeStruct((_NS,), jnp.float32),
            jax.ShapeDtypeStruct((_NS,), jnp.float32),
        ],
    )(px, py, pz, psx, psy, psz)


_CS = 128


def _conv_body(qpos_ref, posT_ref, xr_ref, b_ref, L_ref,
               out_ref, agg_ref, carry_ref, cmin_ref):
    b = pl.program_id(1)

    @pl.when(b == 0)
    def _init():
        agg_ref[...] = jnp.zeros_like(agg_ref)
        carry_ref[...] = jnp.zeros_like(carry_ref)
        cmin_ref[0, 0] = 0.0

    @pl.when(cmin_ref[0, 0] < 32.0)
    def _aggregate():
        q = qpos_ref[...]
        p = posT_ref[...]
        q2 = jnp.sum(q * q, axis=1, keepdims=True)
        p2 = jnp.sum(p * p, axis=0, keepdims=True)
        qp = jnp.dot(q.astype(jnp.bfloat16), p.astype(jnp.bfloat16),
                     preferred_element_type=jnp.float32)
        d2 = (q2 + p2) - 2.0 * qp
        mf = (d2 <= _R2).astype(jnp.float32)

        carry = carry_ref[...]
        parts = []
        run = carry
        for s in range(_C // _CS):
            mfs = mf[:, s * _CS:(s + 1) * _CS]
            excl = jnp.dot(mfs, L_ref[...], preferred_element_type=jnp.float32)
            parts.append(mfs * (run + excl < 32.0).astype(jnp.float32))
            run = run + jnp.sum(mfs, axis=1, keepdims=True)
        A = jnp.concatenate(parts, axis=1).astype(jnp.bfloat16)

        agg_ref[...] += jnp.dot(A, xr_ref[...],
                                preferred_element_type=jnp.float32)
        carry_ref[...] = run
        cmin_ref[0, 0] = jnp.min(run)

    @pl.when(b == _NB - 1)
    def _fin():
        cnt = jnp.minimum(carry_ref[...], 32.0)
        den = jnp.maximum(cnt, 1.0)
        out_ref[...] = agg_ref[...] / den + b_ref[...]


def _conv(qposP, posT8, xr, bias, L):
    return pl.pallas_call(
        _conv_body,
        grid=(_NSP // _TQ, _NB),
        in_specs=[
            pl.BlockSpec((_TQ, 8), lambda t, b: (t, 0)),
            pl.BlockSpec((8, _C), lambda t, b: (0, b)),
            pl.BlockSpec((_C, _F), lambda t, b: (b, 0)),
            pl.BlockSpec((1, _F), lambda t, b: (0, 0)),
            pl.BlockSpec((_CS, _CS), lambda t, b: (0, 0)),
        ],
        out_specs=pl.BlockSpec((_TQ, _F), lambda t, b: (t, 0)),
        out_shape=jax.ShapeDtypeStruct((_NSP, _F), jnp.float32),
        scratch_shapes=[
            pltpu.VMEM((_TQ, _F), jnp.float32),
            pltpu.VMEM((_TQ, 1), jnp.float32),
            pltpu.SMEM((1, 1), jnp.float32),
        ],
    )(qposP, posT8, xr, bias, L)


def kernel(x, pos, batch, W_rel, b_rel, W_root):
    posP = jnp.pad(pos, ((0, _NP - _N), (0, 0)))
    px = posP[:, 0].reshape(_FR, _FC)
    py = posP[:, 1].reshape(_FR, _FC)
    pz = posP[:, 2].reshape(_FR, _FC)

    xP = jnp.pad(x, ((0, _NP - _N), (0, 0)))
    xr, xo = _proj(xP, W_rel, W_root)

    idx, qx, qy, qz = _fps(px, py, pz, posP[:, 0], posP[:, 1], posP[:, 2])
    qpos = jnp.stack([qx, qy, qz], axis=1)

    posT8 = jnp.zeros((8, _NP), jnp.float32)
    posT8 = posT8.at[:3, :].set(posP.T)
    posT8 = posT8.at[0, _N:].set(1e4)

    qposP = jnp.zeros((_NSP, 8), jnp.float32).at[:_NS, :3].set(qpos)
    idxS = jnp.zeros((_NSP,), jnp.int32).at[:_NS].set(idx)

    L = (jnp.arange(_CS, dtype=jnp.int32)[:, None]
         < jnp.arange(_CS, dtype=jnp.int32)[None, :]).astype(jnp.float32)
    bias = b_rel.reshape(1, _F)

    xsel = _sc_gather(xo, idxS)
    outP = _conv(qposP, posT8, xr, bias, L)
    x_out = (outP + xsel)[:_NS]
    qbatch = batch[idx]
    return (x_out, qpos, qbatch, idx)


# speedup vs baseline: 1.3603x; 1.0255x over previous
import functools

import jax
import jax.numpy as jnp
import numpy as np
from jax.experimental import pallas as pl
import jax.experimental.pallas.tpu as pltpu
from jax.experimental.pallas import tpu_sc as plsc

_N = 10000
_NP = 10240
_NS = 2500
_NSP = 2560
_F = 128
_TQ = 256
_C = 2048
_NB = _NP // _C
_R2 = 1.0

_HI = jax.lax.Precision.HIGHEST


def _proj_body(x_ref, wr_ref, wo_ref, xr_ref, xo_ref):
    xb = x_ref[...]
    xr_ref[...] = jnp.dot(xb, wr_ref[...], preferred_element_type=jnp.float32,
                          precision=_HI).astype(jnp.bfloat16)
    xo_ref[...] = jnp.dot(xb, wo_ref[...], preferred_element_type=jnp.float32,
                          precision=_HI)


def _proj(xP, W_rel, W_root):
    blk = 512
    return pl.pallas_call(
        _proj_body,
        grid=(_NP // blk,),
        in_specs=[
            pl.BlockSpec((blk, _F), lambda i: (i, 0)),
            pl.BlockSpec((_F, _F), lambda i: (0, 0)),
            pl.BlockSpec((_F, _F), lambda i: (0, 0)),
        ],
        out_specs=[
            pl.BlockSpec((blk, _F), lambda i: (i, 0)),
            pl.BlockSpec((blk, _F), lambda i: (i, 0)),
        ],
        out_shape=[
            jax.ShapeDtypeStruct((_NP, _F), jnp.bfloat16),
            jax.ShapeDtypeStruct((_NP, _F), jnp.float32),
        ],
    )(xP, W_rel, W_root)


def _sc_gather(xo, idxS):
    info = plsc.get_sparse_core_info()
    nw = info.num_cores * info.num_subcores
    bpw = _NSP // nw
    mesh = plsc.VectorSubcoreMesh(core_axis_name="c", subcore_axis_name="s")

    @functools.partial(
        pl.kernel, mesh=mesh,
        out_type=jax.ShapeDtypeStruct((_NSP, _F), jnp.float32),
        scratch_types=[
            pltpu.VMEM((bpw,), jnp.int32),
            pltpu.VMEM((bpw, _F), jnp.float32),
            pltpu.SemaphoreType.DMA,
        ],
    )
    def k(table_hbm, idx_hbm, out_hbm, idx_v, rows_v, sem):
        wid = (jax.lax.axis_index("s") * info.num_cores
               + jax.lax.axis_index("c"))
        base = wid * bpw
        pltpu.sync_copy(idx_hbm.at[pl.ds(base, bpw)], idx_v)
        pltpu.async_copy(table_hbm.at[idx_v], rows_v, sem).wait()
        pltpu.sync_copy(rows_v, out_hbm.at[pl.ds(base, bpw)])

    return k(xo, idxS)


_FR, _FC = 8, _NP // 8


def _fps_body(px_ref, py_ref, pz_ref, psx_ref, psy_ref, psz_ref,
              idx_ref, qx_ref, qy_ref, qz_ref):
    rows = jax.lax.broadcasted_iota(jnp.int32, (_FR, _FC), 0)
    cols = jax.lax.broadcasted_iota(jnp.int32, (_FR, _FC), 1)
    lin = rows * _FC + cols
    flin = lin.astype(jnp.float32)
    real = lin < _N
    dist0 = jnp.where(real, jnp.inf, -jnp.inf).astype(jnp.float32)

    idx_ref[0] = jnp.int32(0)
    sx, sy, sz = psx_ref[0], psy_ref[0], psz_ref[0]
    qx_ref[0] = sx
    qy_ref[0] = sy
    qz_ref[0] = sz

    def body(i, state):
        dist, sx, sy, sz = state
        dx = px_ref[...] - sx
        dy = py_ref[...] - sy
        dz = pz_ref[...] - sz
        d = (dx * dx + dy * dy) + dz * dz
        dist = jnp.minimum(dist, d)
        m = jnp.max(dist, axis=(0, 1), keepdims=True)
        nxt = jnp.min(jnp.where(dist == m, flin, jnp.float32(3e7))
                      ).astype(jnp.int32)
        sx, sy, sz = psx_ref[nxt], psy_ref[nxt], psz_ref[nxt]
        idx_ref[i] = nxt
        qx_ref[i] = sx
        qy_ref[i] = sy
        qz_ref[i] = sz
        return dist, sx, sy, sz

    jax.lax.fori_loop(1, _NS, body, (dist0, sx, sy, sz))


def _fps(px, py, pz, psx, psy, psz):
    sm = functools.partial(pl.BlockSpec, memory_space=pltpu.SMEM)
    return pl.pallas_call(
        _fps_body,
        in_specs=[pl.BlockSpec((_FR, _FC), lambda: (0, 0))] * 3 + [sm()] * 3,
        out_specs=[sm(), sm(), sm(), sm()],
        out_shape=[
            jax.ShapeDtypeStruct((_NS,), jnp.int32),
            jax.ShapeDtypeStruct((_NS,), jnp.float32),
            jax.ShapeDtypeStruct((_NS,), jnp.float32),
            jax.ShapeDtypeStruct((_NS,), jnp.float32),
        ],
    )(px, py, pz, psx, psy, psz)


_CS = 128


def _conv_body(qpos_ref, posT_ref, xr_ref, b_ref, L_ref,
               out_ref, agg_ref, carry_ref, cmin_ref):
    b = pl.program_id(1)

    @pl.when(b == 0)
    def _init():
        agg_ref[...] = jnp.zeros_like(agg_ref)
        carry_ref[...] = jnp.zeros_like(carry_ref)
        cmin_ref[0, 0] = 0.0

    @pl.when(cmin_ref[0, 0] < 32.0)
    def _aggregate():
        q = qpos_ref[...]
        p = posT_ref[...]
        q2 = jnp.sum(q * q, axis=1, keepdims=True)
        p2 = jnp.sum(p * p, axis=0, keepdims=True)
        qp = jnp.dot(q.astype(jnp.bfloat16), p.astype(jnp.bfloat16),
                     preferred_element_type=jnp.float32)
        d2 = (q2 + p2) - 2.0 * qp
        mf = (d2 <= _R2).astype(jnp.float32)

        carry = carry_ref[...]
        parts = []
        run = carry
        for s in range(_C // _CS):
            mfs = mf[:, s * _CS:(s + 1) * _CS]
            excl = jnp.dot(mfs, L_ref[...], preferred_element_type=jnp.float32)
            parts.append(mfs * (run + excl < 32.0).astype(jnp.float32))
            run = run + jnp.sum(mfs, axis=1, keepdims=True)
        A = jnp.concatenate(parts, axis=1).astype(jnp.bfloat16)

        agg_ref[...] += jnp.dot(A, xr_ref[...],
                                preferred_element_type=jnp.float32)
        carry_ref[...] = run
        cmin_ref[0, 0] = jnp.min(run)

    @pl.when(b == _NB - 1)
    def _fin():
        cnt = jnp.minimum(carry_ref[...], 32.0)
        den = jnp.maximum(cnt, 1.0)
        out_ref[...] = agg_ref[...] / den + b_ref[...]


def _conv(qposP, posT8, xr, bias, L):
    return pl.pallas_call(
        _conv_body,
        grid=(_NSP // _TQ, _NB),
        in_specs=[
            pl.BlockSpec((_TQ, 8), lambda t, b: (t, 0)),
            pl.BlockSpec((8, _C), lambda t, b: (0, b)),
            pl.BlockSpec((_C, _F), lambda t, b: (b, 0)),
            pl.BlockSpec((1, _F), lambda t, b: (0, 0)),
            pl.BlockSpec((_CS, _CS), lambda t, b: (0, 0)),
        ],
        out_specs=pl.BlockSpec((_TQ, _F), lambda t, b: (t, 0)),
        out_shape=jax.ShapeDtypeStruct((_NSP, _F), jnp.float32),
        scratch_shapes=[
            pltpu.VMEM((_TQ, _F), jnp.float32),
            pltpu.VMEM((_TQ, 1), jnp.float32),
            pltpu.SMEM((1, 1), jnp.float32),
        ],
    )(qposP, posT8, xr, bias, L)


def kernel(x, pos, batch, W_rel, b_rel, W_root):
    posP = jnp.pad(pos, ((0, _NP - _N), (0, 0)))
    px = posP[:, 0].reshape(_FR, _FC)
    py = posP[:, 1].reshape(_FR, _FC)
    pz = posP[:, 2].reshape(_FR, _FC)

    xP = jnp.pad(x, ((0, _NP - _N), (0, 0)))
    xr, xo = _proj(xP, W_rel, W_root)

    idx, qx, qy, qz = _fps(px, py, pz, posP[:, 0], posP[:, 1], posP[:, 2])
    qpos = jnp.stack([qx, qy, qz], axis=1)

    posT8 = jnp.zeros((8, _NP), jnp.float32)
    posT8 = posT8.at[:3, :].set(posP.T)
    posT8 = posT8.at[0, _N:].set(1e4)

    qposP = jnp.zeros((_NSP, 8), jnp.float32).at[:_NS, :3].set(qpos)
    idxS = jnp.zeros((_NSP,), jnp.int32).at[:_NS].set(idx)

    L = (jnp.arange(_CS, dtype=jnp.int32)[:, None]
         < jnp.arange(_CS, dtype=jnp.int32)[None, :]).astype(jnp.float32)
    bias = b_rel.reshape(1, _F)

    xsel = _sc_gather(xo, idxS)
    outP = _conv(qposP, posT8, xr, bias, L)
    x_out = (outP + xsel)[:_NS]
    qbatch = batch[idx]
    return (x_out, qpos, qbatch, idx)


# speedup vs baseline: 1.3894x; 1.0214x over previous
import functools

import jax
import jax.numpy as jnp
import numpy as np
from jax.experimental import pallas as pl
import jax.experimental.pallas.tpu as pltpu
from jax.experimental.pallas import tpu_sc as plsc

_N = 10000
_NP = 10240
_NS = 2500
_NSP = 2560
_F = 128
_TQ = 256
_C = 5120
_NB = _NP // _C
_R2 = 1.0

_HI = jax.lax.Precision.HIGHEST


def _proj_body(x_ref, wr_ref, wo_ref, xr_ref, xo_ref):
    xb = x_ref[...]
    xr_ref[...] = jnp.dot(xb, wr_ref[...], preferred_element_type=jnp.float32,
                          precision=_HI).astype(jnp.bfloat16)
    xo_ref[...] = jnp.dot(xb, wo_ref[...], preferred_element_type=jnp.float32,
                          precision=_HI)


def _proj(xP, W_rel, W_root):
    blk = 512
    return pl.pallas_call(
        _proj_body,
        grid=(_NP // blk,),
        in_specs=[
            pl.BlockSpec((blk, _F), lambda i: (i, 0)),
            pl.BlockSpec((_F, _F), lambda i: (0, 0)),
            pl.BlockSpec((_F, _F), lambda i: (0, 0)),
        ],
        out_specs=[
            pl.BlockSpec((blk, _F), lambda i: (i, 0)),
            pl.BlockSpec((blk, _F), lambda i: (i, 0)),
        ],
        out_shape=[
            jax.ShapeDtypeStruct((_NP, _F), jnp.bfloat16),
            jax.ShapeDtypeStruct((_NP, _F), jnp.float32),
        ],
    )(xP, W_rel, W_root)


def _sc_gather(xo, idxS):
    info = plsc.get_sparse_core_info()
    nw = info.num_cores * info.num_subcores
    bpw = _NSP // nw
    mesh = plsc.VectorSubcoreMesh(core_axis_name="c", subcore_axis_name="s")

    @functools.partial(
        pl.kernel, mesh=mesh,
        out_type=jax.ShapeDtypeStruct((_NSP, _F), jnp.float32),
        scratch_types=[
            pltpu.VMEM((bpw,), jnp.int32),
            pltpu.VMEM((bpw, _F), jnp.float32),
            pltpu.SemaphoreType.DMA,
        ],
    )
    def k(table_hbm, idx_hbm, out_hbm, idx_v, rows_v, sem):
        wid = (jax.lax.axis_index("s") * info.num_cores
               + jax.lax.axis_index("c"))
        base = wid * bpw
        pltpu.sync_copy(idx_hbm.at[pl.ds(base, bpw)], idx_v)
        pltpu.async_copy(table_hbm.at[idx_v], rows_v, sem).wait()
        pltpu.sync_copy(rows_v, out_hbm.at[pl.ds(base, bpw)])

    return k(xo, idxS)


_FR, _FC = 8, _NP // 8


def _fps_body(px_ref, py_ref, pz_ref, psx_ref, psy_ref, psz_ref,
              idx_ref, qx_ref, qy_ref, qz_ref):
    rows = jax.lax.broadcasted_iota(jnp.int32, (_FR, _FC), 0)
    cols = jax.lax.broadcasted_iota(jnp.int32, (_FR, _FC), 1)
    lin = rows * _FC + cols
    flin = lin.astype(jnp.float32)
    real = lin < _N
    dist0 = jnp.where(real, jnp.inf, -jnp.inf).astype(jnp.float32)

    idx_ref[0] = jnp.int32(0)
    sx, sy, sz = psx_ref[0], psy_ref[0], psz_ref[0]
    qx_ref[0] = sx
    qy_ref[0] = sy
    qz_ref[0] = sz

    def body(i, state):
        dist, sx, sy, sz = state
        dx = px_ref[...] - sx
        dy = py_ref[...] - sy
        dz = pz_ref[...] - sz
        d = (dx * dx + dy * dy) + dz * dz
        dist = jnp.minimum(dist, d)
        m = jnp.max(dist, axis=(0, 1), keepdims=True)
        nxt = jnp.min(jnp.where(dist == m, flin, jnp.float32(3e7))
                      ).astype(jnp.int32)
        sx, sy, sz = psx_ref[nxt], psy_ref[nxt], psz_ref[nxt]
        idx_ref[i] = nxt
        qx_ref[i] = sx
        qy_ref[i] = sy
        qz_ref[i] = sz
        return dist, sx, sy, sz

    jax.lax.fori_loop(1, _NS, body, (dist0, sx, sy, sz))


def _fps(px, py, pz, psx, psy, psz):
    sm = functools.partial(pl.BlockSpec, memory_space=pltpu.SMEM)
    return pl.pallas_call(
        _fps_body,
        in_specs=[pl.BlockSpec((_FR, _FC), lambda: (0, 0))] * 3 + [sm()] * 3,
        out_specs=[sm(), sm(), sm(), sm()],
        out_shape=[
            jax.ShapeDtypeStruct((_NS,), jnp.int32),
            jax.ShapeDtypeStruct((_NS,), jnp.float32),
            jax.ShapeDtypeStruct((_NS,), jnp.float32),
            jax.ShapeDtypeStruct((_NS,), jnp.float32),
        ],
    )(px, py, pz, psx, psy, psz)


_CS = 128


def _conv_body(qpos_ref, posT_ref, xr_ref, b_ref, L_ref,
               out_ref, agg_ref, carry_ref, cmin_ref):
    b = pl.program_id(1)

    @pl.when(b == 0)
    def _init():
        agg_ref[...] = jnp.zeros_like(agg_ref)
        carry_ref[...] = jnp.zeros_like(carry_ref)
        cmin_ref[0, 0] = 0.0

    @pl.when(cmin_ref[0, 0] < 32.0)
    def _aggregate():
        q = qpos_ref[...]
        p = posT_ref[...]
        q2 = jnp.sum(q * q, axis=1, keepdims=True)
        p2 = jnp.sum(p * p, axis=0, keepdims=True)
        qp = jnp.dot(q.astype(jnp.bfloat16), p.astype(jnp.bfloat16),
                     preferred_element_type=jnp.float32)
        d2 = (q2 + p2) - 2.0 * qp
        mf = (d2 <= _R2).astype(jnp.float32)

        carry = carry_ref[...]
        parts = []
        run = carry
        for s in range(_C // _CS):
            mfs = mf[:, s * _CS:(s + 1) * _CS]
            excl = jnp.dot(mfs, L_ref[...], preferred_element_type=jnp.float32)
            parts.append(mfs * (run + excl < 32.0).astype(jnp.float32))
            run = run + jnp.sum(mfs, axis=1, keepdims=True)
        A = jnp.concatenate(parts, axis=1).astype(jnp.bfloat16)

        agg_ref[...] += jnp.dot(A, xr_ref[...],
                                preferred_element_type=jnp.float32)
        carry_ref[...] = run
        cmin_ref[0, 0] = jnp.min(run)

    @pl.when(b == _NB - 1)
    def _fin():
        cnt = jnp.minimum(carry_ref[...], 32.0)
        den = jnp.maximum(cnt, 1.0)
        out_ref[...] = agg_ref[...] / den + b_ref[...]


def _conv(qposP, posT8, xr, bias, L):
    return pl.pallas_call(
        _conv_body,
        grid=(_NSP // _TQ, _NB),
        in_specs=[
            pl.BlockSpec((_TQ, 8), lambda t, b: (t, 0)),
            pl.BlockSpec((8, _C), lambda t, b: (0, b)),
            pl.BlockSpec((_C, _F), lambda t, b: (b, 0)),
            pl.BlockSpec((1, _F), lambda t, b: (0, 0)),
            pl.BlockSpec((_CS, _CS), lambda t, b: (0, 0)),
        ],
        out_specs=pl.BlockSpec((_TQ, _F), lambda t, b: (t, 0)),
        out_shape=jax.ShapeDtypeStruct((_NSP, _F), jnp.float32),
        scratch_shapes=[
            pltpu.VMEM((_TQ, _F), jnp.float32),
            pltpu.VMEM((_TQ, 1), jnp.float32),
            pltpu.SMEM((1, 1), jnp.float32),
        ],
    )(qposP, posT8, xr, bias, L)


def kernel(x, pos, batch, W_rel, b_rel, W_root):
    posP = jnp.pad(pos, ((0, _NP - _N), (0, 0)))
    px = posP[:, 0].reshape(_FR, _FC)
    py = posP[:, 1].reshape(_FR, _FC)
    pz = posP[:, 2].reshape(_FR, _FC)

    xP = jnp.pad(x, ((0, _NP - _N), (0, 0)))
    xr, xo = _proj(xP, W_rel, W_root)

    idx, qx, qy, qz = _fps(px, py, pz, posP[:, 0], posP[:, 1], posP[:, 2])
    qpos = jnp.stack([qx, qy, qz], axis=1)

    posT8 = jnp.zeros((8, _NP), jnp.float32)
    posT8 = posT8.at[:3, :].set(posP.T)
    posT8 = posT8.at[0, _N:].set(1e4)

    qposP = jnp.zeros((_NSP, 8), jnp.float32).at[:_NS, :3].set(qpos)
    idxS = jnp.zeros((_NSP,), jnp.int32).at[:_NS].set(idx)

    L = (jnp.arange(_CS, dtype=jnp.int32)[:, None]
         < jnp.arange(_CS, dtype=jnp.int32)[None, :]).astype(jnp.float32)
    bias = b_rel.reshape(1, _F)

    xsel = _sc_gather(xo, idxS)
    outP = _conv(qposP, posT8, xr, bias, L)
    x_out = (outP + xsel)[:_NS]
    qbatch = batch[idx]
    return (x_out, qpos, qbatch, idx)
